# baseline (speedup 1.0000x reference)
import functools

import jax
import jax.numpy as jnp
from jax import lax
from jax.experimental import pallas as pl
from jax.experimental.pallas import tpu as pltpu
from jax.experimental.pallas import tpu_sc as plsc

_NC = 2
_NS = 16
_NW = _NC * _NS

_TILE = 128


def _router_body(x_ref, wr_ref, rank_ref, off_ref, cnt_ref, aux_ref):
    T, D = x_ref.shape
    E = wr_ref.shape[0]
    x = x_ref[...]
    wr = wr_ref[...]
    logits = lax.dot_general(
        x, wr, (((1,), (1,)), ((), ())),
        preferred_element_type=jnp.float32,
    )
    m = jnp.max(logits, axis=1, keepdims=True)
    lane = lax.broadcasted_iota(jnp.int32, (T, E), 1)
    eid = jnp.min(jnp.where(logits == m, lane, E), axis=1, keepdims=True)
    onehot = (lane == eid).astype(jnp.float32)

    counts = jnp.sum(onehot, axis=0, keepdims=True)
    cnt8 = jnp.floor((counts + 7.0) * 0.125) * 8.0

    c = cnt8
    k = 1
    while k < E:
        shifted = jnp.concatenate(
            [jnp.zeros((1, k), jnp.float32), c[:, : E - k]], axis=1)
        c = c + shifted
        k *= 2
    offsets = c - cnt8

    p = onehot
    k = 1
    while k < T:
        shifted = jnp.concatenate(
            [jnp.zeros((k, E), jnp.float32), p[: T - k, :]], axis=0)
        p = p + shifted
        k *= 2
    rank = jnp.sum(onehot * (offsets + p - 1.0), axis=1, keepdims=True)
    rank_ref[...] = rank.astype(jnp.int32)
    off_ref[...] = offsets.astype(jnp.int32)
    cnt_ref[...] = counts.astype(jnp.int32)

    probs = jnp.exp(logits - m)
    probs = probs / jnp.sum(probs, axis=1, keepdims=True)
    mean_prob = jnp.sum(probs, axis=0, keepdims=True) * (1.0 / T)
    frac = counts * (1.0 / T)
    aux_ref[...] = E * jnp.sum(frac * mean_prob, axis=1, keepdims=True)


def _swiglu_tile(xb, gw, uw, dw):
    g = lax.dot_general(xb, gw, (((1,), (1,)), ((), ())),
                        preferred_element_type=jnp.float32)
    u = lax.dot_general(xb, uw, (((1,), (1,)), ((), ())),
                        preferred_element_type=jnp.float32)
    h = g * lax.logistic(g) * u
    return lax.dot_general(h, dw, (((1,), (1,)), ((), ())),
                           preferred_element_type=jnp.float32)


def _shared_body(x_ref, wsg_ref, wsu_ref, wsd_ref, y_ref):
    y_ref[...] = _swiglu_tile(
        x_ref[...], wsg_ref[...], wsu_ref[...], wsd_ref[...])


def _moe_body(off_ref, cnt_ref, x_ref, sh_ref, weg_ref, weu_ref, wed_ref,
              y_ref):
    e = pl.program_id(0)
    start = pl.multiple_of(off_ref[e], 8)
    cnt = cnt_ref[e]
    ntiles = (cnt + _TILE - 1) // _TILE

    weg = weg_ref[0]
    weu = weu_ref[0]
    wed = wed_ref[0]

    def body(t, _):
        r0 = start + t * _TILE
        xb = x_ref[pl.ds(r0, _TILE), :]
        y = _swiglu_tile(xb, weg, weu, wed) + sh_ref[pl.ds(r0, _TILE), :]
        y_ref[pl.ds(r0, _TILE), :] = y
        return 0

    lax.fori_loop(0, ntiles, body, 0)


def _dispatch_body(x_hbm, rank_hbm, xs_hbm, idx_v, rows_v, sem):
    ch = idx_v.shape[0]
    wid = lax.axis_index("s") * _NC + lax.axis_index("c")
    base = wid * ch
    pltpu.sync_copy(rank_hbm.at[pl.ds(base, ch)], idx_v)
    pltpu.sync_copy(x_hbm.at[pl.ds(base, ch)], rows_v)
    pltpu.async_copy(rows_v, xs_hbm.at[idx_v], sem).wait()


def _return_body(ys_hbm, rank_hbm, out_hbm, idx_v, rows_v, sem):
    ch = idx_v.shape[0]
    wid = lax.axis_index("s") * _NC + lax.axis_index("c")
    base = wid * ch
    pltpu.sync_copy(rank_hbm.at[pl.ds(base, ch)], idx_v)
    pltpu.async_copy(ys_hbm.at[idx_v], rows_v, sem).wait()
    pltpu.sync_copy(rows_v, out_hbm.at[pl.ds(base, ch)])


def kernel(x, ws_up, ws_gate, ws_down, we_up, we_gate, we_down, wr):
    B, S, D = x.shape
    E, F, _ = we_up.shape
    T = B * S
    TP = ((T + 7) // 8) * 8 + E * 8 + _TILE
    CH = T // _NW

    x_flat = x.reshape(T, D)

    rank2, off2, cnt2, aux = pl.pallas_call(
        _router_body,
        out_shape=(
            jax.ShapeDtypeStruct((T, 1), jnp.int32),
            jax.ShapeDtypeStruct((1, E), jnp.int32),
            jax.ShapeDtypeStruct((1, E), jnp.int32),
            jax.ShapeDtypeStruct((1, 1), jnp.float32),
        ),
    )(x_flat, wr)
    rank = rank2.reshape(T)
    offsets = off2.reshape(E)
    counts = cnt2.reshape(E)

    mesh = plsc.VectorSubcoreMesh(core_axis_name="c", subcore_axis_name="s")
    dispatch = functools.partial(
        pl.kernel,
        mesh=mesh,
        out_type=jax.ShapeDtypeStruct((TP, D), jnp.float32),
        scratch_types=[
            pltpu.VMEM((CH,), jnp.int32),
            pltpu.VMEM((CH, D), jnp.float32),
            pltpu.SemaphoreType.DMA,
        ],
    )(_dispatch_body)
    xs = dispatch(x_flat, rank)

    SH_TILE = 384
    assert TP % SH_TILE == 0
    shared = pl.pallas_call(
        _shared_body,
        grid=(TP // SH_TILE,),
        in_specs=[
            pl.BlockSpec((SH_TILE, D), lambda i: (i, 0)),
            pl.BlockSpec((F, D), lambda i: (0, 0)),
            pl.BlockSpec((F, D), lambda i: (0, 0)),
            pl.BlockSpec((D, F), lambda i: (0, 0)),
        ],
        out_specs=pl.BlockSpec((SH_TILE, D), lambda i: (i, 0)),
        out_shape=jax.ShapeDtypeStruct((TP, D), jnp.float32),
    )(xs, ws_gate, ws_up, ws_down)

    grid_spec = pltpu.PrefetchScalarGridSpec(
        num_scalar_prefetch=2,
        grid=(E,),
        in_specs=[
            pl.BlockSpec((TP, D), lambda e, off, cnt: (0, 0)),
            pl.BlockSpec((TP, D), lambda e, off, cnt: (0, 0)),
            pl.BlockSpec((1, F, D), lambda e, off, cnt: (e, 0, 0)),
            pl.BlockSpec((1, F, D), lambda e, off, cnt: (e, 0, 0)),
            pl.BlockSpec((1, D, F), lambda e, off, cnt: (e, 0, 0)),
        ],
        out_specs=pl.BlockSpec((TP, D), lambda e, off, cnt: (0, 0)),
    )
    ys = pl.pallas_call(
        _moe_body,
        grid_spec=grid_spec,
        out_shape=jax.ShapeDtypeStruct((TP, D), jnp.float32),
        compiler_params=pltpu.CompilerParams(
            vmem_limit_bytes=100 * 1024 * 1024),
    )(offsets, counts, xs, shared, we_gate, we_up, we_down)

    gather = functools.partial(
        pl.kernel,
        mesh=mesh,
        out_type=jax.ShapeDtypeStruct((T, D), jnp.float32),
        scratch_types=[
            pltpu.VMEM((CH,), jnp.int32),
            pltpu.VMEM((CH, D), jnp.float32),
            pltpu.SemaphoreType.DMA,
        ],
    )(_return_body)
    out = gather(ys, rank)

    return out.reshape(B, S, D), aux.reshape(())


# speedup vs baseline: 1.0217x; 1.0217x over previous
import functools

import jax
import jax.numpy as jnp
from jax import lax
from jax.experimental import pallas as pl
from jax.experimental.pallas import tpu as pltpu
from jax.experimental.pallas import tpu_sc as plsc

_NC = 2
_NS = 16
_NW = _NC * _NS

_TILE = 64


def _router_body(x_ref, wr_ref, rank_ref, off_ref, cnt_ref, aux_ref):
    T, D = x_ref.shape
    E = wr_ref.shape[0]
    x = x_ref[...]
    wr = wr_ref[...]
    logits = lax.dot_general(
        x, wr, (((1,), (1,)), ((), ())),
        preferred_element_type=jnp.float32,
    )
    m = jnp.max(logits, axis=1, keepdims=True)
    lane = lax.broadcasted_iota(jnp.int32, (T, E), 1)
    eid = jnp.min(jnp.where(logits == m, lane, E), axis=1, keepdims=True)
    onehot = (lane == eid).astype(jnp.float32)

    counts = jnp.sum(onehot, axis=0, keepdims=True)
    cnt8 = jnp.floor((counts + 7.0) * 0.125) * 8.0

    c = cnt8
    k = 1
    while k < E:
        shifted = jnp.concatenate(
            [jnp.zeros((1, k), jnp.float32), c[:, : E - k]], axis=1)
        c = c + shifted
        k *= 2
    offsets = c - cnt8

    p = onehot
    k = 1
    while k < T:
        shifted = jnp.concatenate(
            [jnp.zeros((k, E), jnp.float32), p[: T - k, :]], axis=0)
        p = p + shifted
        k *= 2
    rank = jnp.sum(onehot * (offsets + p - 1.0), axis=1, keepdims=True)
    rank_ref[...] = rank.astype(jnp.int32)
    off_ref[...] = offsets.astype(jnp.int32)
    cnt_ref[...] = counts.astype(jnp.int32)

    probs = jnp.exp(logits - m)
    probs = probs / jnp.sum(probs, axis=1, keepdims=True)
    mean_prob = jnp.sum(probs, axis=0, keepdims=True) * (1.0 / T)
    frac = counts * (1.0 / T)
    aux_ref[...] = E * jnp.sum(frac * mean_prob, axis=1, keepdims=True)


def _swiglu_tile(xb, gw, uw, dw):
    g = lax.dot_general(xb, gw, (((1,), (1,)), ((), ())),
                        preferred_element_type=jnp.float32)
    u = lax.dot_general(xb, uw, (((1,), (1,)), ((), ())),
                        preferred_element_type=jnp.float32)
    h = g * lax.logistic(g) * u
    return lax.dot_general(h, dw, (((1,), (1,)), ((), ())),
                           preferred_element_type=jnp.float32)


def _shared_body(x_ref, wsg_ref, wsu_ref, wsd_ref, y_ref):
    y_ref[...] = _swiglu_tile(
        x_ref[...], wsg_ref[...], wsu_ref[...], wsd_ref[...])


def _moe_body(off_ref, cnt_ref, x_ref, sh_ref, weg_ref, weu_ref, wed_ref,
              y_ref):
    e = pl.program_id(0)
    start = pl.multiple_of(off_ref[e], 8)
    cnt = cnt_ref[e]
    ntiles = (cnt + _TILE - 1) // _TILE

    weg = weg_ref[0]
    weu = weu_ref[0]
    wed = wed_ref[0]

    def body(t, _):
        r0 = start + t * _TILE
        xb = x_ref[pl.ds(r0, _TILE), :]
        y = _swiglu_tile(xb, weg, weu, wed) + sh_ref[pl.ds(r0, _TILE), :]
        y_ref[pl.ds(r0, _TILE), :] = y
        return 0

    lax.fori_loop(0, ntiles, body, 0)


def _dispatch_body(x_hbm, rank_hbm, xs_hbm, idx_v, rows_v, sem):
    ch = idx_v.shape[0]
    wid = lax.axis_index("s") * _NC + lax.axis_index("c")
    base = wid * ch
    pltpu.sync_copy(rank_hbm.at[pl.ds(base, ch)], idx_v)
    pltpu.sync_copy(x_hbm.at[pl.ds(base, ch)], rows_v)
    pltpu.async_copy(rows_v, xs_hbm.at[idx_v], sem).wait()


def _return_body(ys_hbm, rank_hbm, out_hbm, idx_v, rows_v, sem):
    ch = idx_v.shape[0]
    wid = lax.axis_index("s") * _NC + lax.axis_index("c")
    base = wid * ch
    pltpu.sync_copy(rank_hbm.at[pl.ds(base, ch)], idx_v)
    pltpu.async_copy(ys_hbm.at[idx_v], rows_v, sem).wait()
    pltpu.sync_copy(rows_v, out_hbm.at[pl.ds(base, ch)])


def kernel(x, ws_up, ws_gate, ws_down, we_up, we_gate, we_down, wr):
    B, S, D = x.shape
    E, F, _ = we_up.shape
    T = B * S
    TP = ((T + 7) // 8) * 8 + E * 8 + 128
    CH = T // _NW

    x_flat = x.reshape(T, D)

    rank2, off2, cnt2, aux = pl.pallas_call(
        _router_body,
        out_shape=(
            jax.ShapeDtypeStruct((T, 1), jnp.int32),
            jax.ShapeDtypeStruct((1, E), jnp.int32),
            jax.ShapeDtypeStruct((1, E), jnp.int32),
            jax.ShapeDtypeStruct((1, 1), jnp.float32),
        ),
    )(x_flat, wr)
    rank = rank2.reshape(T)
    offsets = off2.reshape(E)
    counts = cnt2.reshape(E)

    mesh = plsc.VectorSubcoreMesh(core_axis_name="c", subcore_axis_name="s")
    dispatch = functools.partial(
        pl.kernel,
        mesh=mesh,
        out_type=jax.ShapeDtypeStruct((TP, D), jnp.float32),
        scratch_types=[
            pltpu.VMEM((CH,), jnp.int32),
            pltpu.VMEM((CH, D), jnp.float32),
            pltpu.SemaphoreType.DMA,
        ],
    )(_dispatch_body)
    xs = dispatch(x_flat, rank)

    SH_TILE = 384
    assert TP % SH_TILE == 0
    shared = pl.pallas_call(
        _shared_body,
        grid=(TP // SH_TILE,),
        in_specs=[
            pl.BlockSpec((SH_TILE, D), lambda i: (i, 0)),
            pl.BlockSpec((F, D), lambda i: (0, 0)),
            pl.BlockSpec((F, D), lambda i: (0, 0)),
            pl.BlockSpec((D, F), lambda i: (0, 0)),
        ],
        out_specs=pl.BlockSpec((SH_TILE, D), lambda i: (i, 0)),
        out_shape=jax.ShapeDtypeStruct((TP, D), jnp.float32),
    )(xs, ws_gate, ws_up, ws_down)

    grid_spec = pltpu.PrefetchScalarGridSpec(
        num_scalar_prefetch=2,
        grid=(E,),
        in_specs=[
            pl.BlockSpec((TP, D), lambda e, off, cnt: (0, 0)),
            pl.BlockSpec((TP, D), lambda e, off, cnt: (0, 0)),
            pl.BlockSpec((1, F, D), lambda e, off, cnt: (e, 0, 0)),
            pl.BlockSpec((1, F, D), lambda e, off, cnt: (e, 0, 0)),
            pl.BlockSpec((1, D, F), lambda e, off, cnt: (e, 0, 0)),
        ],
        out_specs=pl.BlockSpec((TP, D), lambda e, off, cnt: (0, 0)),
    )
    ys = pl.pallas_call(
        _moe_body,
        grid_spec=grid_spec,
        out_shape=jax.ShapeDtypeStruct((TP, D), jnp.float32),
        compiler_params=pltpu.CompilerParams(
            vmem_limit_bytes=100 * 1024 * 1024),
    )(offsets, counts, xs, shared, we_gate, we_up, we_down)

    gather = functools.partial(
        pl.kernel,
        mesh=mesh,
        out_type=jax.ShapeDtypeStruct((T, D), jnp.float32),
        scratch_types=[
            pltpu.VMEM((CH,), jnp.int32),
            pltpu.VMEM((CH, D), jnp.float32),
            pltpu.SemaphoreType.DMA,
        ],
    )(_return_body)
    out = gather(ys, rank)

    return out.reshape(B, S, D), aux.reshape(())


# speedup vs baseline: 1.0254x; 1.0037x over previous
import functools

import jax
import jax.numpy as jnp
from jax import lax
from jax.experimental import pallas as pl
from jax.experimental.pallas import tpu as pltpu
from jax.experimental.pallas import tpu_sc as plsc

_NC = 2
_NS = 16
_NW = _NC * _NS

_TILE = 64


def _router_body(x_ref, wr_ref, rank_ref, ept_ref, aux_ref):
    T, D = x_ref.shape
    E = wr_ref.shape[0]
    NT = T // _TILE + E
    x = x_ref[...]
    wr = wr_ref[...]
    logits = lax.dot_general(
        x, wr, (((1,), (1,)), ((), ())),
        preferred_element_type=jnp.float32,
    )
    m = jnp.max(logits, axis=1, keepdims=True)
    lane = lax.broadcasted_iota(jnp.int32, (T, E), 1)
    eid = jnp.min(jnp.where(logits == m, lane, E), axis=1, keepdims=True)
    onehot = (lane == eid).astype(jnp.float32)

    counts = jnp.sum(onehot, axis=0, keepdims=True)
    ntiles = jnp.floor((counts + (_TILE - 1.0)) * (1.0 / _TILE))

    c = ntiles
    k = 1
    while k < E:
        shifted = jnp.concatenate(
            [jnp.zeros((1, k), jnp.float32), c[:, : E - k]], axis=1)
        c = c + shifted
        k *= 2
    tstart = c - ntiles
    row_off = tstart * float(_TILE)
    total_tiles = c[:, E - 1:E]

    p = onehot
    k = 1
    while k < T:
        shifted = jnp.concatenate(
            [jnp.zeros((k, E), jnp.float32), p[: T - k, :]], axis=0)
        p = p + shifted
        k *= 2
    rank = jnp.sum(onehot * (row_off + p - 1.0), axis=1, keepdims=True)
    rank_ref[...] = rank.astype(jnp.int32)

    ti = lax.broadcasted_iota(jnp.int32, (NT, E), 0).astype(jnp.float32)
    lane_f = lax.broadcasted_iota(jnp.int32, (NT, E), 1).astype(jnp.float32)
    mask = jnp.logical_and(ti >= tstart, ti < tstart + ntiles)
    maskf = mask.astype(jnp.float32)
    ept = jnp.sum(lane_f * maskf, axis=1, keepdims=True)
    anyf = jnp.sum(maskf, axis=1, keepdims=True)
    lane1 = lax.broadcasted_iota(jnp.int32, (1, E), 1).astype(jnp.float32)
    lastexp = jnp.max(lane1 * (ntiles > 0.0).astype(jnp.float32),
                      axis=1, keepdims=True)
    ept = ept + (1.0 - anyf) * lastexp
    ept_ref[...] = jnp.concatenate([ept, total_tiles], axis=0).astype(
        jnp.int32)

    probs = jnp.exp(logits - m)
    probs = probs / jnp.sum(probs, axis=1, keepdims=True)
    mean_prob = jnp.sum(probs, axis=0, keepdims=True) * (1.0 / T)
    frac = counts * (1.0 / T)
    aux_ref[...] = E * jnp.sum(frac * mean_prob, axis=1, keepdims=True)


def _swiglu_tile(xb, gw, uw, dw):
    g = lax.dot_general(xb, gw, (((1,), (1,)), ((), ())),
                        preferred_element_type=jnp.float32)
    u = lax.dot_general(xb, uw, (((1,), (1,)), ((), ())),
                        preferred_element_type=jnp.float32)
    h = g * lax.logistic(g) * u
    return lax.dot_general(h, dw, (((1,), (1,)), ((), ())),
                           preferred_element_type=jnp.float32)


def _shared_body(x_ref, wsg_ref, wsu_ref, wsd_ref, y_ref):
    y_ref[...] = _swiglu_tile(
        x_ref[...], wsg_ref[...], wsu_ref[...], wsd_ref[...])


def _moe_body(ept_ref, x_ref, weg_ref, weu_ref, wed_ref, y_ref):
    t = pl.program_id(0)
    total_tiles = ept_ref[ept_ref.shape[0] - 1]

    @pl.when(t < total_tiles)
    def _():
        y_ref[...] = _swiglu_tile(
            x_ref[...], weg_ref[0], weu_ref[0], wed_ref[0])


def _dispatch_body(x_hbm, rank_hbm, xs_hbm, idx_v, rows_v, sem):
    ch = idx_v.shape[0]
    wid = lax.axis_index("s") * _NC + lax.axis_index("c")
    base = wid * ch
    pltpu.sync_copy(rank_hbm.at[pl.ds(base, ch)], idx_v)
    pltpu.sync_copy(x_hbm.at[pl.ds(base, ch)], rows_v)
    pltpu.async_copy(rows_v, xs_hbm.at[idx_v], sem).wait()


def _return_body(ys_hbm, rank_hbm, sh_hbm, out_hbm, idx_v, rows_v, sh_v, sem):
    ch = idx_v.shape[0]
    d = rows_v.shape[1]
    wid = lax.axis_index("s") * _NC + lax.axis_index("c")
    base = wid * ch
    pltpu.sync_copy(rank_hbm.at[pl.ds(base, ch)], idx_v)
    cp = pltpu.async_copy(ys_hbm.at[idx_v], rows_v, sem)
    pltpu.sync_copy(sh_hbm.at[pl.ds(base, ch)], sh_v)
    cp.wait()

    def row(r, carry):
        for c in range(d // 16):
            sl = pl.ds(c * 16, 16)
            rows_v[r, sl] = rows_v[r, sl] + sh_v[r, sl]
        return carry

    lax.fori_loop(0, ch, row, 0)
    pltpu.sync_copy(rows_v, out_hbm.at[pl.ds(base, ch)])


def kernel(x, ws_up, ws_gate, ws_down, we_up, we_gate, we_down, wr):
    B, S, D = x.shape
    E, F, _ = we_up.shape
    T = B * S
    NT = T // _TILE + E
    TP = NT * _TILE
    CH = T // _NW

    x_flat = x.reshape(T, D)

    rank2, ept2, aux = pl.pallas_call(
        _router_body,
        out_shape=(
            jax.ShapeDtypeStruct((T, 1), jnp.int32),
            jax.ShapeDtypeStruct((NT + 1, 1), jnp.int32),
            jax.ShapeDtypeStruct((1, 1), jnp.float32),
        ),
    )(x_flat, wr)
    rank = rank2.reshape(T)
    ept = ept2.reshape(NT + 1)

    mesh = plsc.VectorSubcoreMesh(core_axis_name="c", subcore_axis_name="s")
    dispatch = functools.partial(
        pl.kernel,
        mesh=mesh,
        out_type=jax.ShapeDtypeStruct((TP, D), jnp.float32),
        scratch_types=[
            pltpu.VMEM((CH,), jnp.int32),
            pltpu.VMEM((CH, D), jnp.float32),
            pltpu.SemaphoreType.DMA,
        ],
    )(_dispatch_body)
    xs = dispatch(x_flat, rank)

    SH_TILE = 256
    shared = pl.pallas_call(
        _shared_body,
        grid=(T // SH_TILE,),
        in_specs=[
            pl.BlockSpec((SH_TILE, D), lambda i: (i, 0)),
            pl.BlockSpec((F, D), lambda i: (0, 0)),
            pl.BlockSpec((F, D), lambda i: (0, 0)),
            pl.BlockSpec((D, F), lambda i: (0, 0)),
        ],
        out_specs=pl.BlockSpec((SH_TILE, D), lambda i: (i, 0)),
        out_shape=jax.ShapeDtypeStruct((T, D), jnp.float32),
    )(x_flat, ws_gate, ws_up, ws_down)

    grid_spec = pltpu.PrefetchScalarGridSpec(
        num_scalar_prefetch=1,
        grid=(NT,),
        in_specs=[
            pl.BlockSpec((_TILE, D), lambda t, ept: (t, 0)),
            pl.BlockSpec((1, F, D), lambda t, ept: (ept[t], 0, 0)),
            pl.BlockSpec((1, F, D), lambda t, ept: (ept[t], 0, 0)),
            pl.BlockSpec((1, D, F), lambda t, ept: (ept[t], 0, 0)),
        ],
        out_specs=pl.BlockSpec((_TILE, D), lambda t, ept: (t, 0)),
    )
    ys = pl.pallas_call(
        _moe_body,
        grid_spec=grid_spec,
        out_shape=jax.ShapeDtypeStruct((TP, D), jnp.float32),
    )(ept, xs, we_gate, we_up, we_down)

    gather = functools.partial(
        pl.kernel,
        mesh=mesh,
        out_type=jax.ShapeDtypeStruct((T, D), jnp.float32),
        scratch_types=[
            pltpu.VMEM((CH,), jnp.int32),
            pltpu.VMEM((CH, D), jnp.float32),
            pltpu.VMEM((CH, D), jnp.float32),
            pltpu.SemaphoreType.DMA,
        ],
    )(_return_body)
    out = gather(ys, rank, shared)

    return out.reshape(B, S, D), aux.reshape(())


# speedup vs baseline: 1.1109x; 1.0833x over previous
import functools

import jax
import jax.numpy as jnp
from jax import lax
from jax.experimental import pallas as pl
from jax.experimental.pallas import tpu as pltpu
from jax.experimental.pallas import tpu_sc as plsc

_NC = 2
_NS = 16
_NW = _NC * _NS

_TILE = 64


def _router_body(x_ref, wr_ref, rank_ref, ept_ref, aux_ref):
    T, D = x_ref.shape
    E = wr_ref.shape[0]
    NT = T // _TILE + E
    x = x_ref[...]
    wr = wr_ref[...]
    logits = lax.dot_general(
        x, wr, (((1,), (1,)), ((), ())),
        preferred_element_type=jnp.float32,
    )
    m = jnp.max(logits, axis=1, keepdims=True)
    lane = lax.broadcasted_iota(jnp.int32, (T, E), 1)
    eid = jnp.min(jnp.where(logits == m, lane, E), axis=1, keepdims=True)
    onehot = (lane == eid).astype(jnp.float32)

    counts = jnp.sum(onehot, axis=0, keepdims=True)
    ntiles = jnp.floor((counts + (_TILE - 1.0)) * (1.0 / _TILE))

    c = ntiles
    k = 1
    while k < E:
        shifted = jnp.concatenate(
            [jnp.zeros((1, k), jnp.float32), c[:, : E - k]], axis=1)
        c = c + shifted
        k *= 2
    tstart = c - ntiles
    row_off = tstart * float(_TILE)
    total_tiles = c[:, E - 1:E]

    p = onehot
    k = 1
    while k < T:
        shifted = jnp.concatenate(
            [jnp.zeros((k, E), jnp.float32), p[: T - k, :]], axis=0)
        p = p + shifted
        k *= 2
    rank = jnp.sum(onehot * (row_off + p - 1.0), axis=1, keepdims=True)
    rank_ref[...] = rank.astype(jnp.int32)

    ti = lax.broadcasted_iota(jnp.int32, (NT, E), 0).astype(jnp.float32)
    lane_f = lax.broadcasted_iota(jnp.int32, (NT, E), 1).astype(jnp.float32)
    mask = jnp.logical_and(ti >= tstart, ti < tstart + ntiles)
    maskf = mask.astype(jnp.float32)
    ept = jnp.sum(lane_f * maskf, axis=1, keepdims=True)
    anyf = jnp.sum(maskf, axis=1, keepdims=True)
    lane1 = lax.broadcasted_iota(jnp.int32, (1, E), 1).astype(jnp.float32)
    lastexp = jnp.max(lane1 * (ntiles > 0.0).astype(jnp.float32),
                      axis=1, keepdims=True)
    ept = ept + (1.0 - anyf) * lastexp
    ept_ref[...] = jnp.concatenate([ept, total_tiles], axis=0).astype(
        jnp.int32)

    probs = jnp.exp(logits - m)
    probs = probs / jnp.sum(probs, axis=1, keepdims=True)
    mean_prob = jnp.sum(probs, axis=0, keepdims=True) * (1.0 / T)
    frac = counts * (1.0 / T)
    aux_ref[...] = E * jnp.sum(frac * mean_prob, axis=1, keepdims=True)


def _swiglu_tile(xb, gw, uw, dw):
    g = lax.dot_general(xb, gw, (((1,), (1,)), ((), ())),
                        preferred_element_type=jnp.float32)
    u = lax.dot_general(xb, uw, (((1,), (1,)), ((), ())),
                        preferred_element_type=jnp.float32)
    h = g * lax.logistic(g) * u
    return lax.dot_general(h, dw, (((1,), (1,)), ((), ())),
                           preferred_element_type=jnp.float32)


def _shared_body(x_ref, wsg_ref, wsu_ref, wsd_ref, y_ref):
    y_ref[...] = _swiglu_tile(
        x_ref[...], wsg_ref[...], wsu_ref[...], wsd_ref[...])


def _moe_body(ept_ref, x_ref, weg_ref, weu_ref, wed_ref, y_ref):
    t = pl.program_id(0)
    total_tiles = ept_ref[ept_ref.shape[0] - 1]

    @pl.when(t < total_tiles)
    def _():
        y_ref[...] = _swiglu_tile(
            x_ref[...], weg_ref[0], weu_ref[0], wed_ref[0])


def _dispatch_body(x_hbm, rank_hbm, xs_hbm, idx_v, rows_v, sem):
    ch = idx_v.shape[0]
    wid = lax.axis_index("s") * _NC + lax.axis_index("c")
    base = wid * ch
    pltpu.sync_copy(rank_hbm.at[pl.ds(base, ch)], idx_v)
    pltpu.sync_copy(x_hbm.at[pl.ds(base, ch)], rows_v)
    pltpu.async_copy(rows_v, xs_hbm.at[idx_v], sem).wait()


def _return_body(ys_hbm, rank_hbm, sh_hbm, out_hbm, idx_v, rows_v, sh_v, sem):
    ch = idx_v.shape[0]
    d = rows_v.shape[1]
    wid = lax.axis_index("s") * _NC + lax.axis_index("c")
    base = wid * ch
    pltpu.sync_copy(rank_hbm.at[pl.ds(base, ch)], idx_v)
    cp = pltpu.async_copy(ys_hbm.at[idx_v], rows_v, sem)
    pltpu.sync_copy(sh_hbm.at[pl.ds(base, ch)], sh_v)
    cp.wait()

    def row(r, carry):
        for c in range(d // 16):
            sl = pl.ds(c * 16, 16)
            rows_v[r, sl] = rows_v[r, sl] + sh_v[r, sl]
        return carry

    lax.fori_loop(0, ch, row, 0)
    pltpu.sync_copy(rows_v, out_hbm.at[pl.ds(base, ch)])


def kernel(x, ws_up, ws_gate, ws_down, we_up, we_gate, we_down, wr):
    B, S, D = x.shape
    E, F, _ = we_up.shape
    T = B * S
    NT = T // _TILE + E
    TP = NT * _TILE
    CH = T // _NW

    x_flat = x.reshape(T, D)

    rank2, ept2, aux = pl.pallas_call(
        _router_body,
        out_shape=(
            jax.ShapeDtypeStruct((T, 1), jnp.int32),
            jax.ShapeDtypeStruct((NT + 1, 1), jnp.int32),
            jax.ShapeDtypeStruct((1, 1), jnp.float32),
        ),
    )(x_flat, wr)
    rank = rank2.reshape(T)
    ept = ept2.reshape(NT + 1)

    SH_TILE = 512
    shared = pl.pallas_call(
        _shared_body,
        grid=(T // SH_TILE,),
        in_specs=[
            pl.BlockSpec((SH_TILE, D), lambda i: (i, 0)),
            pl.BlockSpec((F, D), lambda i: (0, 0)),
            pl.BlockSpec((F, D), lambda i: (0, 0)),
            pl.BlockSpec((D, F), lambda i: (0, 0)),
        ],
        out_specs=pl.BlockSpec((SH_TILE, D), lambda i: (i, 0)),
        out_shape=jax.ShapeDtypeStruct((T, D), jnp.float32),
    )(x_flat, ws_gate, ws_up, ws_down)

    mesh = plsc.VectorSubcoreMesh(core_axis_name="c", subcore_axis_name="s")
    dispatch = functools.partial(
        pl.kernel,
        mesh=mesh,
        out_type=jax.ShapeDtypeStruct((TP, D), jnp.float32),
        scratch_types=[
            pltpu.VMEM((CH,), jnp.int32),
            pltpu.VMEM((CH, D), jnp.float32),
            pltpu.SemaphoreType.DMA,
        ],
    )(_dispatch_body)
    xs = dispatch(x_flat, rank)


    def _tile_idx(t, ept):
        return (jnp.minimum(t, ept[NT] - 1), 0)

    grid_spec = pltpu.PrefetchScalarGridSpec(
        num_scalar_prefetch=1,
        grid=(NT,),
        in_specs=[
            pl.BlockSpec((_TILE, D), _tile_idx),
            pl.BlockSpec((1, F, D), lambda t, ept: (ept[t], 0, 0)),
            pl.BlockSpec((1, F, D), lambda t, ept: (ept[t], 0, 0)),
            pl.BlockSpec((1, D, F), lambda t, ept: (ept[t], 0, 0)),
        ],
        out_specs=pl.BlockSpec((_TILE, D), _tile_idx),
    )
    ys = pl.pallas_call(
        _moe_body,
        grid_spec=grid_spec,
        out_shape=jax.ShapeDtypeStruct((TP, D), jnp.float32),
    )(ept, xs, we_gate, we_up, we_down)

    gather = functools.partial(
        pl.kernel,
        mesh=mesh,
        out_type=jax.ShapeDtypeStruct((T, D), jnp.float32),
        scratch_types=[
            pltpu.VMEM((CH,), jnp.int32),
            pltpu.VMEM((CH, D), jnp.float32),
            pltpu.VMEM((CH, D), jnp.float32),
            pltpu.SemaphoreType.DMA,
        ],
    )(_return_body)
    out = gather(ys, rank, shared)

    return out.reshape(B, S, D), aux.reshape(())
